# Initial kernel scaffold; baseline (speedup 1.0000x reference)
import jax
import jax.numpy as jnp
from jax.experimental import pallas as pl


def kernel(label_a, label_t, label_v, label_r):
    raise NotImplementedError("write your pallas kernel here")


# speedup vs baseline: 165.2934x; 165.2934x over previous
import functools

import jax
import jax.numpy as jnp
from jax.experimental import pallas as pl

CA = 527
CV = 1000
SEG = 256
ROIS = 8
K_TOP = 13
KA = 5
KV = 10
CAP = 528
CVP = 1024
RU = 272


def _passmask(data, levels_ref):
    rows, cols = data.shape
    m = jnp.max(data, axis=1, keepdims=True)
    iota = jax.lax.broadcasted_iota(jnp.int32, (rows, cols), 1)
    work = data
    passm = jnp.zeros((rows, cols), jnp.float32)
    for k in range(K_TOP):
        mk = jnp.max(work, axis=1, keepdims=True)
        first = jnp.min(jnp.where(work == mk, iota, cols), axis=1, keepdims=True)
        sel = iota == first
        ok = (mk / m) * levels_ref[0, k] >= 0.5
        passm = jnp.where(sel & ok, 1.0, passm)
        work = jnp.where(sel, -jnp.inf, work)
    return passm


def _score_body(levels_ref, x_ref, out_ref):
    b = pl.program_id(0)
    pm = _passmask(x_ref[...], levels_ref)

    @pl.when(b < 8)
    def _():
        out_ref[0] = jnp.max(pm.reshape(32, ROIS, CVP), axis=1)

    @pl.when(b == 8)
    def _():
        out_ref[0] = jnp.zeros((32, CVP), jnp.float32)
        out_ref[0, 0:1] = jnp.max(pm, axis=0, keepdims=True)


def _mask_body(vseg_ref, lt_ref, la_ref, va_ref, aa_ref):
    va_ref[0:SEG] = vseg_ref[0:8].reshape(SEG, CVP)
    va_ref[SEG:SEG + 1] = vseg_ref[8, 0:1]
    va_ref[SEG + 1:RU] = jnp.zeros((RU - SEG - 1, CVP), jnp.float32)
    lt = lt_ref[...]
    thr = jnp.minimum(jnp.float32(0.4), jnp.max(lt, axis=1, keepdims=True))
    aa_ref[0:SEG] = (lt >= thr).astype(jnp.float32)
    la = la_ref[...]
    thrf = jnp.minimum(jnp.float32(0.4), jnp.max(la))
    aa_ref[SEG:SEG + 1] = (la[0:1] >= thrf).astype(jnp.float32)
    aa_ref[SEG + 1:RU] = jnp.zeros((RU - SEG - 1, CAP), jnp.float32)


def _firstk(mask, k):
    n = mask.shape[1]
    ri = jax.lax.broadcasted_iota(jnp.int32, (n, n), 0)
    ci = jax.lax.broadcasted_iota(jnp.int32, (n, n), 1)
    tri = (ri <= ci).astype(jnp.float32)
    cum = jax.lax.dot_general(mask, tri, (((1,), (0,)), ((), ())),
                              preferred_element_type=jnp.float32)
    return jnp.where(cum <= k, mask, 0.0)


def _accum_body(am_ref, vm_ref, sa_ref, sv_ref, ca_ref, cv_ref, cav_ref):
    a = _firstk(am_ref[...], KA)
    v = _firstk(vm_ref[...], KV)
    sa_ref[...] = jnp.sum(a, axis=0, keepdims=True)
    sv_ref[...] = jnp.sum(v, axis=0, keepdims=True)
    gram = lambda x, y: jax.lax.dot_general(
        x, y, (((0,), (0,)), ((), ())), preferred_element_type=jnp.float32)
    ca_ref[...] = gram(a, a)
    cv_ref[...] = gram(v, v)
    cav_ref[...] = gram(a, v)


@jax.jit
def kernel(label_a, label_t, label_v, label_r):
    levels = jnp.power(jnp.float32(0.95),
                       jnp.arange(1.0, 17.0, dtype=jnp.float32))[None, :]
    cat = jnp.concatenate([label_r, label_v], axis=0)
    cat = jnp.pad(cat, ((0, 0), (0, CVP - CV)))

    vseg = pl.pallas_call(
        _score_body,
        grid=(9,),
        in_specs=[
            pl.BlockSpec((1, 16), lambda b: (0, 0)),
            pl.BlockSpec((SEG, CVP), lambda b: (b, 0)),
        ],
        out_specs=pl.BlockSpec((1, 32, CVP), lambda b: (b, 0, 0)),
        out_shape=jax.ShapeDtypeStruct((9, 32, CVP), jnp.float32),
    )(levels, cat)

    lt_p = jnp.pad(label_t, ((0, 0), (0, CAP - CA)))
    la_p = jnp.pad(label_a, ((0, 0), (0, CAP - CA)))
    vmask, amask = pl.pallas_call(
        _mask_body,
        out_shape=(
            jax.ShapeDtypeStruct((RU, CVP), jnp.float32),
            jax.ShapeDtypeStruct((RU, CAP), jnp.float32),
        ),
    )(vseg, lt_p, la_p)

    sa, sv, ca, cv, cav = pl.pallas_call(
        _accum_body,
        out_shape=(
            jax.ShapeDtypeStruct((1, CAP), jnp.float32),
            jax.ShapeDtypeStruct((1, CVP), jnp.float32),
            jax.ShapeDtypeStruct((CAP, CAP), jnp.float32),
            jax.ShapeDtypeStruct((CVP, CVP), jnp.float32),
            jax.ShapeDtypeStruct((CAP, CVP), jnp.float32),
        ),
    )(amask, vmask)

    return (sa[0, :CA], sv[0, :CV], ca[:CA, :CA], cv[:CV, :CV], cav[:CA, :CV])
